# baseline (speedup 1.0000x reference)
import functools
import jax
import jax.numpy as jnp
import numpy as np
from jax import lax
from jax.experimental import pallas as pl
from jax.experimental.pallas import tpu as pltpu

NUM_CLASSES_K = 1000
EPS_K = 0.1
RATIO_K = 0.6
BATCH_K = 16384
CHUNK = 2048
SUB = 512
SPC = CHUNK // SUB
NCHUNK = BATCH_K // CHUNK
NSTEP = BATCH_K // SUB
ROWS = SUB // 128
TOPK = int(BATCH_K * RATIO_K)
MININT = np.int32(-2147483648)
MAXPOS = np.int32(2147483647)


def _ce_kernel(x_hbm, t_ref, o_ref, bufs, sems, ps_ref):
    i = pl.program_id(0)
    chunk = i // SPC
    sub = i % SPC
    slot = chunk % 2

    def copy(c, sl):
        return pltpu.make_async_copy(
            x_hbm.at[pl.ds(c * CHUNK, CHUNK), :], bufs.at[sl], sems.at[sl]
        )

    @pl.when(i == 0)
    def _prime():
        copy(0, 0).start()
        copy(1, 1).start()

    @pl.when(sub == 0)
    def _wait():
        copy(chunk, slot).wait()

    x = bufs[slot, pl.ds(sub * SUB, SUB), :]
    t = t_ref[0, 0, :]
    m = jnp.max(x, axis=1)
    se = jnp.sum(jnp.exp(x - m[:, None]), axis=1)
    lse = m + jnp.log(se)
    cols = lax.broadcasted_iota(jnp.int32, x.shape, 1)
    w_hi = (1.0 - EPS_K) + EPS_K / NUM_CLASSES_K
    w_lo = EPS_K / NUM_CLASSES_K
    r = jnp.sum(x * jnp.where(cols == t[:, None], w_hi, w_lo), axis=1)
    ps = (lse - r) / NUM_CLASSES_K
    ps_ref[pl.ds(i * ROWS, ROWS), :] = ps.reshape(ROWS, 128)

    @pl.when((sub == SPC - 1) & (chunk < NCHUNK - 2))
    def _refill():
        copy(chunk + 2, slot).start()

    @pl.when(i == NSTEP - 1)
    def _epilogue():
        v = ps_ref[...]
        b = lax.bitcast_convert_type(v, jnp.int32)
        skey = b ^ (jnp.right_shift(b, 31) & MAXPOS)

        def body(tstep, p):
            bit = jnp.left_shift(jnp.int32(1), 31 - tstep)
            cand = p | bit
            cnt = jnp.sum((skey >= (cand ^ MININT)).astype(jnp.int32))
            return jnp.where(cnt >= TOPK, cand, p)

        p = lax.fori_loop(0, 32, body, jnp.int32(0))
        skey_k = p ^ MININT
        bk = jnp.where(skey_k >= 0, skey_k, skey_k ^ MAXPOS)
        v_k = lax.bitcast_convert_type(bk, jnp.float32)
        gt = skey > skey_k
        cnt_gt = jnp.sum(gt.astype(jnp.int32))
        sum_gt = jnp.sum(jnp.where(gt, v, 0.0))
        loss = (sum_gt + (TOPK - cnt_gt).astype(jnp.float32) * v_k) / TOPK
        o_ref[...] = loss.reshape(1, 1)


@jax.jit
def kernel(inputs, targets):
    t3 = targets.astype(jnp.int32).reshape(NSTEP, 1, SUB)
    out = pl.pallas_call(
        _ce_kernel,
        grid=(NSTEP,),
        in_specs=[
            pl.BlockSpec(memory_space=pl.ANY),
            pl.BlockSpec((1, 1, SUB), lambda i: (i, 0, 0)),
        ],
        out_specs=pl.BlockSpec((1, 1), lambda i: (0, 0)),
        out_shape=jax.ShapeDtypeStruct((1, 1), jnp.float32),
        scratch_shapes=[
            pltpu.VMEM((2, CHUNK, NUM_CLASSES_K), jnp.float32),
            pltpu.SemaphoreType.DMA((2,)),
            pltpu.VMEM((128, 128), jnp.float32),
        ],
        compiler_params=pltpu.CompilerParams(
            dimension_semantics=("arbitrary",),
        ),
    )(inputs, t3)
    return out[0, 0]
